# Initial kernel scaffold; baseline (speedup 1.0000x reference)
import jax
import jax.numpy as jnp
from jax.experimental import pallas as pl


def kernel(pred, inds, ind_mask, gt):
    raise NotImplementedError("write your pallas kernel here")


# speedup vs baseline: 1.0059x; 1.0059x over previous
import functools

import jax
import jax.numpy as jnp
from jax import lax
from jax.experimental import pallas as pl
from jax.experimental.pallas import tpu as pltpu
from jax.experimental.pallas import tpu_sc as plsc

NC = 2
NS = 16
L = 16
NW = NC * NS
K = 500
KPAD = 512
ROWS_PER_W = 32
HW = 128 * 128


def _sc_body(pred_hbm, inds_hbm, mask_hbm, gt_hbm, out_hbm,
             idx_v, m_v, gt0, gt1, row0, row1, out_v, rsem, gsem):
    cid = lax.axis_index("c")
    sid = lax.axis_index("s")
    wid = sid * NC + cid
    b = wid // 2
    c0 = (wid % 2) * ROWS_PER_W
    gt_bufs = (gt0, gt1)
    row_bufs = (row0, row1)

    zf = jnp.zeros((L,), jnp.float32)

    pltpu.sync_copy(inds_hbm.at[b], idx_v)
    pltpu.sync_copy(mask_hbm.at[b], m_v)

    def start(j):
        p = j % 2
        rc = pltpu.make_async_copy(pred_hbm.at[b, c0 + j], row_bufs[p],
                                   rsem.at[p])
        gc = pltpu.make_async_copy(gt_hbm.at[b, c0 + j], gt_bufs[p],
                                   gsem.at[p])
        rc.start()
        gc.start()
        return rc, gc

    pend = [start(0), start(1)]

    def m_step(t, a):
        return a + m_v[pl.ds(t * L, L)]
    acc_m = lax.fori_loop(0, KPAD // L, m_step, zf) * float(ROWS_PER_W)

    def chunk(row_ref, gt_ref):
        def step(t, a):
            iv = idx_v[pl.ds(t * L, L)]
            g = plsc.load_gather(row_ref, [iv])
            m = m_v[pl.ds(t * L, L)]
            gv = gt_ref[pl.ds(t * L, L)]
            return a + jnp.abs(g * m - gv * m)
        return step

    acc = zf
    for j in range(ROWS_PER_W):
        p = j % 2
        rc, gc = pend[p]
        rc.wait()
        gc.wait()
        acc = lax.fori_loop(0, KPAD // L, chunk(row_bufs[p], gt_bufs[p]), acc)
        if j + 2 < ROWS_PER_W:
            pend[p] = start(j + 2)

    out_v[pl.ds(0, L)] = acc
    out_v[pl.ds(L, L)] = acc_m
    pltpu.sync_copy(out_v, out_hbm.at[wid])


@jax.jit
def _run(pred_flat, inds32, ind_mask, gt):
    mesh = plsc.VectorSubcoreMesh(core_axis_name="c", subcore_axis_name="s",
                                  num_cores=NC, num_subcores=NS)
    f = pl.kernel(
        _sc_body,
        out_type=jax.ShapeDtypeStruct((NW, 2 * L), jnp.float32),
        mesh=mesh,
        scratch_types=[
            pltpu.VMEM((KPAD,), jnp.int32),
            pltpu.VMEM((KPAD,), jnp.float32),
            pltpu.VMEM((KPAD,), jnp.float32),
            pltpu.VMEM((KPAD,), jnp.float32),
            pltpu.VMEM((HW,), jnp.float32),
            pltpu.VMEM((HW,), jnp.float32),
            pltpu.VMEM((2 * L,), jnp.float32),
            pltpu.SemaphoreType.DMA((2,)),
            pltpu.SemaphoreType.DMA((2,)),
        ],
        compiler_params=pltpu.CompilerParams(needs_layout_passes=False),
    )
    out = f(pred_flat, inds32, ind_mask, gt)
    return out[:, :L].sum() / (out[:, L:].sum() + 0.0001)


def kernel(pred, inds, ind_mask, gt):
    batch, c, h, w = pred.shape
    pred_flat = pred.reshape(batch, c, h * w)
    k = inds.shape[1]
    pad = KPAD - k
    inds_p = jnp.pad(inds.astype(jnp.int32), ((0, 0), (0, pad)))
    mask_p = jnp.pad(ind_mask, ((0, 0), (0, pad)))
    gt_p = jnp.pad(gt, ((0, 0), (0, 0), (0, pad)))
    return _run(pred_flat, inds_p, mask_p, gt_p)


# speedup vs baseline: 2.1901x; 2.1773x over previous
import functools

import jax
import jax.numpy as jnp
from jax import lax
from jax.experimental import pallas as pl
from jax.experimental.pallas import tpu as pltpu
from jax.experimental.pallas import tpu_sc as plsc

NC = 2
NS = 16
L = 16
NW = NC * NS
K = 500
KPAD = 512
ROWS_PER_W = 32
HW = 128 * 128


def _sc_body(pred_hbm, inds_hbm, mask_hbm, gt_hbm, out_hbm,
             idx_v, m_v, gt0, gt1, row0, row1, out_v, rsem, gsem):
    cid = lax.axis_index("c")
    sid = lax.axis_index("s")
    wid = sid * NC + cid
    b = wid // 2
    c0 = (wid % 2) * ROWS_PER_W
    gt_bufs = (gt0, gt1)
    row_bufs = (row0, row1)

    zf = jnp.zeros((L,), jnp.float32)

    pltpu.sync_copy(inds_hbm.at[b], idx_v)
    pltpu.sync_copy(mask_hbm.at[b], m_v)

    def start(j):
        p = j % 2
        rc = pltpu.make_async_copy(pred_hbm.at[b, c0 + j], row_bufs[p],
                                   rsem.at[p])
        gc = pltpu.make_async_copy(gt_hbm.at[b, c0 + j], gt_bufs[p],
                                   gsem.at[p])
        rc.start()
        gc.start()
        return rc, gc

    pend = [start(0), start(1)]

    def m_step(t, a):
        return a + m_v[pl.ds(t * L, L)]
    acc_m = lax.fori_loop(0, KPAD // L, m_step, zf) * float(ROWS_PER_W)

    def chunk(row_ref, gt_ref):
        def step(t, a):
            iv = idx_v[pl.ds(t * L, L)]
            ih = lax.shift_right_logical(iv, 7)
            iw = lax.bitwise_and(iv, 127)
            g = plsc.load_gather(row_ref, [ih, iw])
            m = m_v[pl.ds(t * L, L)]
            gv = gt_ref[pl.ds(t * L, L)]
            return a + jnp.abs(g * m - gv * m)
        return step

    acc = zf
    for j in range(ROWS_PER_W):
        p = j % 2
        rc, gc = pend[p]
        rc.wait()
        gc.wait()
        acc = lax.fori_loop(0, KPAD // L, chunk(row_bufs[p], gt_bufs[p]), acc)
        if j + 2 < ROWS_PER_W:
            pend[p] = start(j + 2)

    out_v[pl.ds(0, L)] = acc
    out_v[pl.ds(L, L)] = acc_m
    pltpu.sync_copy(out_v, out_hbm.at[wid])


@jax.jit
def _run(pred4, inds32, ind_mask, gt):
    mesh = plsc.VectorSubcoreMesh(core_axis_name="c", subcore_axis_name="s",
                                  num_cores=NC, num_subcores=NS)
    f = pl.kernel(
        _sc_body,
        out_type=jax.ShapeDtypeStruct((NW, 2 * L), jnp.float32),
        mesh=mesh,
        scratch_types=[
            pltpu.VMEM((KPAD,), jnp.int32),
            pltpu.VMEM((KPAD,), jnp.float32),
            pltpu.VMEM((KPAD,), jnp.float32),
            pltpu.VMEM((KPAD,), jnp.float32),
            pltpu.VMEM((128, 128), jnp.float32),
            pltpu.VMEM((128, 128), jnp.float32),
            pltpu.VMEM((2 * L,), jnp.float32),
            pltpu.SemaphoreType.DMA((2,)),
            pltpu.SemaphoreType.DMA((2,)),
        ],
        compiler_params=pltpu.CompilerParams(needs_layout_passes=False),
    )
    out = f(pred4, inds32, ind_mask, gt)
    return out[:, :L].sum() / (out[:, L:].sum() + 0.0001)


def kernel(pred, inds, ind_mask, gt):
    k = inds.shape[1]
    pad = KPAD - k
    inds_p = jnp.pad(inds.astype(jnp.int32), ((0, 0), (0, pad)))
    mask_p = jnp.pad(ind_mask, ((0, 0), (0, pad)))
    gt_p = jnp.pad(gt, ((0, 0), (0, 0), (0, pad)))
    return _run(pred, inds_p, mask_p, gt_p)


# speedup vs baseline: 2.5768x; 1.1766x over previous
import jax
import jax.numpy as jnp
from jax import lax
from jax.experimental import pallas as pl
from jax.experimental.pallas import tpu as pltpu
from jax.experimental.pallas import tpu_sc as plsc

NC = 2
NS = 16
L = 16
NW = NC * NS
K = 500
KPAD = 512
ROWS_PER_W = 32
HW = 128 * 128
TOTW = ROWS_PER_W * KPAD
GCH = 128
NDMA = TOTW // GCH
CPD = GCH // L


def _sc_body(pred_hbm, inds_hbm, mask_hbm, gt_hbm, out_hbm,
             idx_v, m_v, ibuf, dbuf, gtbuf, out_v, gsem, ssem):
    cid = lax.axis_index("c")
    sid = lax.axis_index("s")
    wid = sid * NC + cid
    b = wid // 2
    c0 = (wid % 2) * ROWS_PER_W
    zf = jnp.zeros((L,), jnp.float32)

    pltpu.sync_copy(inds_hbm.at[b], idx_v)
    pltpu.sync_copy(mask_hbm.at[b], m_v)
    gt_copies = []
    for j in range(ROWS_PER_W):
        gc = pltpu.make_async_copy(gt_hbm.at[b, c0 + j],
                                   gtbuf.at[pl.ds(j * KPAD, KPAD)], ssem)
        gc.start()
        gt_copies.append(gc)

    base0 = (b * 64 + c0) * HW

    def build(ct, _):
        t = lax.rem(ct, KPAD // L)
        j = ct // (KPAD // L)
        base = base0 + j * HW
        ibuf[pl.ds(ct * L, L)] = idx_v[pl.ds(t * L, L)] + base
        return 0
    lax.fori_loop(0, TOTW // L, build, 0)

    def fire(r, _):
        pltpu.make_async_copy(
            pred_hbm.at[ibuf.at[pl.ds(r * GCH, GCH)]],
            dbuf.at[pl.ds(r * GCH, GCH)], gsem).start()
        return 0
    lax.fori_loop(0, NDMA, fire, 0)

    for gc in gt_copies:
        gc.wait()

    def drain(r, acc):
        pltpu.make_async_copy(
            pred_hbm.at[ibuf.at[pl.ds(0, GCH)]],
            dbuf.at[pl.ds(0, GCH)], gsem).wait()
        for u in range(CPD):
            ct = r * CPD + u
            t = lax.rem(ct, KPAD // L)
            g = dbuf[pl.ds(ct * L, L)]
            m = m_v[pl.ds(t * L, L)]
            gv = gtbuf[pl.ds(ct * L, L)]
            acc = acc + jnp.abs(g * m - gv * m)
        return acc
    acc = lax.fori_loop(0, NDMA, drain, zf)

    def m_step(t, a):
        return a + m_v[pl.ds(t * L, L)]
    acc_m = lax.fori_loop(0, KPAD // L, m_step, zf) * float(ROWS_PER_W)

    out_v[pl.ds(0, L)] = acc
    out_v[pl.ds(L, L)] = acc_m
    pltpu.sync_copy(out_v, out_hbm.at[wid])


@jax.jit
def _run(pred_flat, inds32, ind_mask, gt):
    mesh = plsc.VectorSubcoreMesh(core_axis_name="c", subcore_axis_name="s",
                                  num_cores=NC, num_subcores=NS)
    f = pl.kernel(
        _sc_body,
        out_type=jax.ShapeDtypeStruct((NW, 2 * L), jnp.float32),
        mesh=mesh,
        scratch_types=[
            pltpu.VMEM((KPAD,), jnp.int32),
            pltpu.VMEM((KPAD,), jnp.float32),
            pltpu.VMEM((TOTW,), jnp.int32),
            pltpu.VMEM((TOTW,), jnp.float32),
            pltpu.VMEM((TOTW,), jnp.float32),
            pltpu.VMEM((2 * L,), jnp.float32),
            pltpu.SemaphoreType.DMA,
            pltpu.SemaphoreType.DMA,
        ],
        compiler_params=pltpu.CompilerParams(needs_layout_passes=False),
    )
    out = f(pred_flat, inds32, ind_mask, gt)
    return out[:, :L].sum() / (out[:, L:].sum() + 0.0001)


def kernel(pred, inds, ind_mask, gt):
    k = inds.shape[1]
    pad = KPAD - k
    inds_p = jnp.pad(inds.astype(jnp.int32), ((0, 0), (0, pad)))
    mask_p = jnp.pad(ind_mask, ((0, 0), (0, pad)))
    gt_p = jnp.pad(gt, ((0, 0), (0, 0), (0, pad)))
    return _run(pred.reshape(-1), inds_p, mask_p, gt_p)


# speedup vs baseline: 2.7801x; 1.0789x over previous
import jax
import jax.numpy as jnp
from jax import lax
from jax.experimental import pallas as pl
from jax.experimental.pallas import tpu as pltpu
from jax.experimental.pallas import tpu_sc as plsc

NC = 2
NS = 16
L = 16
NW = NC * NS
K = 500
KPAD = 512
ROWS_PER_W = 32
HW = 128 * 128
TOTW = ROWS_PER_W * KPAD
GCH = 128
NDMA = TOTW // GCH
CPD = GCH // L


def _sc_body(pred_hbm, inds_hbm, mask_hbm, gt_hbm, out_hbm,
             idx_v, m_v, ibuf, dbuf, gtbuf, out_v, gsem, ssem):
    cid = lax.axis_index("c")
    sid = lax.axis_index("s")
    wid = sid * NC + cid
    b = wid // 2
    c0 = (wid % 2) * ROWS_PER_W
    zf = jnp.zeros((L,), jnp.float32)

    pltpu.sync_copy(inds_hbm.at[b], idx_v)
    pltpu.sync_copy(mask_hbm.at[b], m_v)
    gt_copies = []
    for j in range(ROWS_PER_W):
        gc = pltpu.make_async_copy(gt_hbm.at[b, c0 + j],
                                   gtbuf.at[pl.ds(j * KPAD, KPAD)], ssem)
        gc.start()
        gt_copies.append(gc)

    base0 = (b * 64 + c0) * HW

    def build_row(j, _):
        base = base0 + j * HW
        o = j * KPAD

        def build(t, _):
            ibuf[pl.ds(o + t * L, L)] = idx_v[pl.ds(t * L, L)] + base
            return 0
        lax.fori_loop(0, KPAD // L, build, 0)

        def fire(r, _):
            pltpu.make_async_copy(
                pred_hbm.at[ibuf.at[pl.ds(o + r * GCH, GCH)]],
                dbuf.at[pl.ds(o + r * GCH, GCH)], gsem).start()
            return 0
        lax.fori_loop(0, KPAD // GCH, fire, 0)
        return 0
    lax.fori_loop(0, ROWS_PER_W, build_row, 0)

    for gc in gt_copies:
        gc.wait()

    def drain(r, acc):
        pltpu.make_async_copy(
            pred_hbm.at[ibuf.at[pl.ds(0, GCH)]],
            dbuf.at[pl.ds(0, GCH)], gsem).wait()
        for u in range(CPD):
            ct = r * CPD + u
            t = lax.rem(ct, KPAD // L)
            g = dbuf[pl.ds(ct * L, L)]
            m = m_v[pl.ds(t * L, L)]
            gv = gtbuf[pl.ds(ct * L, L)]
            acc = acc + jnp.abs(g * m - gv * m)
        return acc
    acc = lax.fori_loop(0, NDMA, drain, zf)

    def m_step(t, a):
        return a + m_v[pl.ds(t * L, L)]
    acc_m = lax.fori_loop(0, KPAD // L, m_step, zf) * float(ROWS_PER_W)

    out_v[pl.ds(0, L)] = acc
    out_v[pl.ds(L, L)] = acc_m
    pltpu.sync_copy(out_v, out_hbm.at[wid])


@jax.jit
def _run(pred_flat, inds32, ind_mask, gt):
    mesh = plsc.VectorSubcoreMesh(core_axis_name="c", subcore_axis_name="s",
                                  num_cores=NC, num_subcores=NS)
    f = pl.kernel(
        _sc_body,
        out_type=jax.ShapeDtypeStruct((NW, 2 * L), jnp.float32),
        mesh=mesh,
        scratch_types=[
            pltpu.VMEM((KPAD,), jnp.int32),
            pltpu.VMEM((KPAD,), jnp.float32),
            pltpu.VMEM((TOTW,), jnp.int32),
            pltpu.VMEM((TOTW,), jnp.float32),
            pltpu.VMEM((TOTW,), jnp.float32),
            pltpu.VMEM((2 * L,), jnp.float32),
            pltpu.SemaphoreType.DMA,
            pltpu.SemaphoreType.DMA,
        ],
        compiler_params=pltpu.CompilerParams(needs_layout_passes=False),
    )
    out = f(pred_flat, inds32, ind_mask, gt)
    return out[:, :L].sum() / (out[:, L:].sum() + 0.0001)


def kernel(pred, inds, ind_mask, gt):
    k = inds.shape[1]
    pad = KPAD - k
    inds_p = jnp.pad(inds.astype(jnp.int32), ((0, 0), (0, pad)))
    mask_p = jnp.pad(ind_mask, ((0, 0), (0, pad)))
    gt_p = jnp.pad(gt, ((0, 0), (0, 0), (0, pad)))
    return _run(pred.reshape(-1), inds_p, mask_p, gt_p)
